# Initial kernel scaffold; baseline (speedup 1.0000x reference)
import jax
import jax.numpy as jnp
from jax.experimental import pallas as pl


def kernel(node_energy, node_attrs, ptr, edge_index, batch, node_level, scale, shift):
    raise NotImplementedError("write your pallas kernel here")


# speedup vs baseline: 68.5209x; 68.5209x over previous
import functools

import jax
import jax.numpy as jnp
from jax import lax
from jax.experimental import pallas as pl
from jax.experimental.pallas import tpu as pltpu
from jax.experimental.pallas import tpu_sc as plsc

N = 100000
E = 1600000
G = 256
Z = 10

NC = 2
NS = 16
NW = NC * NS

EW = E // NW
ECHUNK = 10000
NITER_E = ECHUNK // 16

NODES_W = 3136
NBLK = NODES_W // 16

_mesh = plsc.VectorSubcoreMesh(core_axis_name="c", subcore_axis_name="s")
_params = pltpu.CompilerParams(needs_layout_passes=False)


@functools.partial(
    pl.kernel,
    mesh=_mesh,
    compiler_params=_params,
    out_type=jax.ShapeDtypeStruct((NW * G,), jnp.float32),
    scratch_types=[
        pltpu.VMEM((N,), jnp.int32),
        pltpu.VMEM((ECHUNK,), jnp.int32),
        pltpu.VMEM((G,), jnp.float32),
    ],
)
def _edge_flags(edge_dst, batch_hbm, flags_out, batch_v, dst_v, flag_v):
    wid = lax.axis_index("s") * NC + lax.axis_index("c")
    base = wid * EW
    pltpu.sync_copy(batch_hbm, batch_v)
    zero16 = jnp.zeros((16,), jnp.float32)
    for b in range(G // 16):
        flag_v[pl.ds(b * 16, 16)] = zero16
    one16 = jnp.ones((16,), jnp.float32)

    for k in range(EW // ECHUNK):
        pltpu.sync_copy(edge_dst.at[pl.ds(base + k * ECHUNK, ECHUNK)], dst_v)

        def body(i, carry):
            idx16 = dst_v[pl.ds(i * 16, 16)]
            g16 = plsc.load_gather(batch_v, [idx16])
            plsc.store_scatter(flag_v, [g16], one16)
            return carry

        lax.fori_loop(0, NITER_E, body, 0)

    pltpu.sync_copy(flag_v, flags_out.at[pl.ds(wid * G, G)])


@functools.partial(
    pl.kernel,
    mesh=_mesh,
    compiler_params=_params,
    out_type=jax.ShapeDtypeStruct((N,), jnp.float32),
    scratch_types=[
        pltpu.VMEM((NW * G,), jnp.float32),
        pltpu.VMEM((G,), jnp.int32),
        pltpu.VMEM((G,), jnp.int32),
        pltpu.VMEM((G,), jnp.float32),
        pltpu.VMEM((2 * Z * 16,), jnp.float32),
        pltpu.VMEM((2 * Z * 16,), jnp.float32),
        pltpu.VMEM((NODES_W,), jnp.float32),
        pltpu.VMEM((NODES_W,), jnp.int32),
        pltpu.VMEM((NODES_W,), jnp.int32),
        pltpu.VMEM((NODES_W * Z,), jnp.float32),
        pltpu.VMEM((NODES_W,), jnp.float32),
    ],
)
def _node_energy(flags_hbm, plo_hbm, phi_hbm, energy_hbm, batch_hbm,
                 level_hbm, attrs_hbm, scaleb_hbm, shiftb_hbm, out_hbm,
                 flags_v, plo_v, phi_v, mask_v, sc_v, sh_v,
                 en_v, bat_v, lev_v, att_v, res_v):
    wid = lax.axis_index("s") * NC + lax.axis_index("c")
    base = jnp.minimum(wid * NODES_W, N - NODES_W)
    pltpu.sync_copy(flags_hbm, flags_v)
    pltpu.sync_copy(plo_hbm, plo_v)
    pltpu.sync_copy(phi_hbm, phi_v)
    pltpu.sync_copy(scaleb_hbm, sc_v)
    pltpu.sync_copy(shiftb_hbm, sh_v)
    pltpu.sync_copy(energy_hbm.at[pl.ds(base, NODES_W)], en_v)
    pltpu.sync_copy(batch_hbm.at[pl.ds(base, NODES_W)], bat_v)
    pltpu.sync_copy(level_hbm.at[pl.ds(base, NODES_W)], lev_v)
    pltpu.sync_copy(attrs_hbm.at[pl.ds(base * Z, NODES_W * Z)], att_v)

    zero16 = jnp.zeros((16,), jnp.float32)
    for b in range(G // 16):
        off = b * 16

        def racc(r, acc, off=off):
            return acc + flags_v[pl.ds(r * G + off, 16)]

        edges = lax.fori_loop(0, NW, racc, zero16)
        nn = phi_v[pl.ds(off, 16)] - plo_v[pl.ds(off, 16)]
        iso = jnp.logical_and(nn == 1, edges == 0.0)
        mask_v[pl.ds(off, 16)] = jnp.where(iso, 0.0, 1.0)

    s0 = [sc_v[pl.ds(z * 16, 16)] for z in range(Z)]
    s1 = [sc_v[pl.ds((Z + z) * 16, 16)] for z in range(Z)]
    t0 = [sh_v[pl.ds(z * 16, 16)] for z in range(Z)]
    t1 = [sh_v[pl.ds((Z + z) * 16, 16)] for z in range(Z)]
    iota = lax.iota(jnp.int32, 16)

    def nblock(j, carry):
        off = j * 16
        e16 = en_v[pl.ds(off, 16)]
        b16 = bat_v[pl.ds(off, 16)]
        l16 = lev_v[pl.ds(off, 16)]
        m16 = plsc.load_gather(mask_v, [b16])
        lm = l16 == 0
        rowb = (iota + off) * Z
        s = zero16
        t = zero16
        for z in range(Z):
            a_z = plsc.load_gather(att_v, [rowb + z])
            s = s + a_z * jnp.where(lm, s0[z], s1[z])
            t = t + a_z * jnp.where(lm, t0[z], t1[z])
        res_v[pl.ds(off, 16)] = m16 * (e16 * s + t)
        return carry

    lax.fori_loop(0, NBLK, nblock, 0)
    pltpu.sync_copy(res_v, out_hbm.at[pl.ds(base, NODES_W)])


def kernel(node_energy, node_attrs, ptr, edge_index, batch, node_level,
           scale, shift):
    edge_dst = edge_index[1]
    flags = _edge_flags(edge_dst, batch)
    scale_b = jnp.broadcast_to(scale[:, :, None], (2, Z, 16)).reshape(-1)
    shift_b = jnp.broadcast_to(shift[:, :, None], (2, Z, 16)).reshape(-1)
    attrs_flat = node_attrs.reshape(-1)
    return _node_energy(flags, ptr[:-1], ptr[1:], node_energy, batch,
                        node_level, attrs_flat, scale_b, shift_b)


# speedup vs baseline: 76.2280x; 1.1125x over previous
import functools

import jax
import jax.numpy as jnp
from jax import lax
from jax.experimental import pallas as pl
from jax.experimental.pallas import tpu as pltpu
from jax.experimental.pallas import tpu_sc as plsc

N = 100000
E = 1600000
G = 256
Z = 10

NC = 2
NS = 16
NW = NC * NS

ES = 50048
ECH = 2176
NCH = ES // ECH
UNROLL = 8
EBODIES = ECH // (16 * UNROLL)

NODES_W = 3136
NBLK = NODES_W // 16

_mesh = plsc.VectorSubcoreMesh(core_axis_name="c", subcore_axis_name="s")
_params = pltpu.CompilerParams(needs_layout_passes=False,
                               use_tc_tiling_on_sc=False)


@functools.partial(
    pl.kernel,
    mesh=_mesh,
    compiler_params=_params,
    out_type=jax.ShapeDtypeStruct((NW * G,), jnp.float32),
    scratch_types=[
        pltpu.VMEM((N,), jnp.int32),
        pltpu.VMEM((2, ECH), jnp.int32),
        pltpu.VMEM((G,), jnp.float32),
    ],
)
def _edge_flags(edge_index, batch_hbm, flags_out, batch_v, dst_v, flag_v):
    wid = lax.axis_index("s") * NC + lax.axis_index("c")
    base = jnp.minimum(wid * ES, E - ES)
    pltpu.sync_copy(batch_hbm, batch_v)
    zero16 = jnp.zeros((16,), jnp.float32)
    for b in range(G // 16):
        flag_v[pl.ds(b * 16, 16)] = zero16
    one16 = jnp.ones((16,), jnp.float32)

    for k in range(NCH):
        pltpu.sync_copy(edge_index.at[:, pl.ds(base + k * ECH, ECH)], dst_v)

        def body(i, carry):
            for u in range(UNROLL):
                off = (i * UNROLL + u) * 16
                idx16 = dst_v[1, pl.ds(off, 16)]
                g16 = plsc.load_gather(batch_v, [idx16])
                plsc.store_scatter(flag_v, [g16], one16)
            return carry

        lax.fori_loop(0, EBODIES, body, 0)

    pltpu.sync_copy(flag_v, flags_out.at[pl.ds(wid * G, G)])


@functools.partial(
    pl.kernel,
    mesh=_mesh,
    compiler_params=_params,
    out_type=jax.ShapeDtypeStruct((N,), jnp.float32),
    scratch_types=[
        pltpu.VMEM((NW * G,), jnp.float32),
        pltpu.VMEM((G + 1,), jnp.int32),
        pltpu.VMEM((G,), jnp.float32),
        pltpu.VMEM((2, Z), jnp.float32),
        pltpu.VMEM((2, Z), jnp.float32),
        pltpu.VMEM((NODES_W,), jnp.float32),
        pltpu.VMEM((NODES_W,), jnp.int32),
        pltpu.VMEM((NODES_W,), jnp.int32),
        pltpu.VMEM((NODES_W, Z), jnp.float32),
        pltpu.VMEM((NODES_W,), jnp.float32),
    ],
)
def _node_energy(flags_hbm, ptr_hbm, energy_hbm, batch_hbm,
                 level_hbm, attrs_hbm, scale_hbm, shift_hbm, out_hbm,
                 flags_v, ptr_v, mask_v, sc_v, sh_v,
                 en_v, bat_v, lev_v, att_v, res_v):
    wid = lax.axis_index("s") * NC + lax.axis_index("c")
    base = jnp.minimum(wid * NODES_W, N - NODES_W)
    pltpu.sync_copy(flags_hbm, flags_v)
    pltpu.sync_copy(ptr_hbm, ptr_v)
    pltpu.sync_copy(scale_hbm, sc_v)
    pltpu.sync_copy(shift_hbm, sh_v)
    pltpu.sync_copy(energy_hbm.at[pl.ds(base, NODES_W)], en_v)
    pltpu.sync_copy(batch_hbm.at[pl.ds(base, NODES_W)], bat_v)
    pltpu.sync_copy(level_hbm.at[pl.ds(base, NODES_W)], lev_v)
    pltpu.sync_copy(attrs_hbm.at[pl.ds(base, NODES_W), :], att_v)

    zero16 = jnp.zeros((16,), jnp.float32)
    for b in range(G // 16):
        off = b * 16

        def racc(r, acc, off=off):
            return acc + flags_v[pl.ds(r * G + off, 16)]

        edges = lax.fori_loop(0, NW, racc, zero16)
        iota16 = lax.iota(jnp.int32, 16)
        nn = plsc.load_gather(ptr_v, [iota16 + (off + 1)]) - ptr_v[pl.ds(off, 16)]
        iso = jnp.logical_and(nn == 1, edges == 0.0)
        mask_v[pl.ds(off, 16)] = jnp.where(iso, 0.0, 1.0)

    zi = jnp.zeros((16,), jnp.int32)
    iota = lax.iota(jnp.int32, 16)

    def nblock(j, carry):
        off = j * 16
        e16 = en_v[pl.ds(off, 16)]
        b16 = bat_v[pl.ds(off, 16)]
        l16 = lev_v[pl.ds(off, 16)]
        m16 = plsc.load_gather(mask_v, [b16])
        row16 = iota + off
        s = zero16
        t = zero16
        for z in range(Z):
            a_z = plsc.load_gather(att_v, [row16, zi + z])
            s = s + a_z * plsc.load_gather(sc_v, [l16, zi + z])
            t = t + a_z * plsc.load_gather(sh_v, [l16, zi + z])
        res_v[pl.ds(off, 16)] = m16 * (e16 * s + t)
        return carry

    lax.fori_loop(0, NBLK, nblock, 0)
    pltpu.sync_copy(res_v, out_hbm.at[pl.ds(base, NODES_W)])


def kernel(node_energy, node_attrs, ptr, edge_index, batch, node_level,
           scale, shift):
    flags = _edge_flags(edge_index, batch)
    return _node_energy(flags, ptr, node_energy, batch, node_level,
                        node_attrs, scale, shift)
